# baseline (speedup 1.0000x reference)
import jax
import jax.numpy as jnp
from jax.experimental import pallas as pl
from jax.experimental.pallas import tpu as pltpu


def _dequant_body(s_ref, c_ref, o_ref):
    c = c_ref[0].astype(jnp.bfloat16)
    s = s_ref[0, 0].astype(jnp.bfloat16)
    o_ref[0, 0] = jax.lax.dot_general(
        c, s, (((0,), (1,)), ((), ())),
        preferred_element_type=jnp.float32)


def kernel(sample, codebook):
    n, m, h, w, k = sample.shape
    d = codebook.shape[-1]
    hw = h * w
    s = sample.reshape(n, m, hw, k)
    out = pl.pallas_call(
        _dequant_body,
        grid=(n, m),
        in_specs=[
            pl.BlockSpec((1, 1, hw, k), lambda ni, mi: (ni, mi, 0, 0)),
            pl.BlockSpec((1, k, d), lambda ni, mi: (mi, 0, 0)),
        ],
        out_specs=pl.BlockSpec((1, 1, d, hw), lambda ni, mi: (ni, mi, 0, 0)),
        out_shape=jax.ShapeDtypeStruct((n, m, d, hw), jnp.float32),
    )(s, codebook)
    return out.reshape(n, m * d, h, w)
